# baseline (speedup 1.0000x reference)
import functools

import numpy as np

import jax
import jax.numpy as jnp
from jax.experimental import pallas as pl
from jax.experimental.pallas import tpu as pltpu

_BB = 64
_VMEM_LIMIT = 100 * 1024 * 1024


def _fused_body(xh_ref, t1_ref, t2_ref, t2b_ref, wf1_ref, wf2_ref,
                b1_ref, b2_ref, bf1_ref, bf2_ref, o_ref):
    BB = xh_ref.shape[1]
    M1 = 28 * BB
    M2 = 14 * BB
    f32 = jnp.float32
    bf16 = jnp.bfloat16

    a1 = jnp.concatenate(
        [xh_ref[d:d + 28].reshape(M1, 32) for d in range(3)], axis=1)
    p1b = []
    for ci in range(4):
        nw = 256 if ci < 3 else 128
        acc = jnp.dot(a1, t1_ref[:, 256 * ci:256 * ci + nw],
                      preferred_element_type=f32)
        u = acc.reshape(14, 2, BB, nw)
        u = jnp.maximum(u[:, 0], u[:, 1])
        h = nw // 2
        u = jnp.maximum(u[:, :, 0:h], u[:, :, h:nw])
        p1b.append(u)
    v = jnp.concatenate(p1b, axis=2)
    v = jnp.maximum(v + b1_ref[...].reshape(1, 1, 448), 0.0)
    p1 = v.astype(bf16)

    zc = jnp.zeros((14, BB, 32), bf16)
    zr = jnp.zeros((1, BB, 512), bf16)
    x2 = jnp.concatenate([zc, p1, zc], axis=2)
    x2 = jnp.concatenate([zr, x2, zr], axis=0)

    outs = []
    for ni in range(4):
        kw = 192 if ni < 3 else 128
        nw = 256 if ni < 3 else 128
        acc = None
        for d in range(3):
            av = x2[d:d + 14].reshape(M2, 512)[:, 128 * ni:128 * ni + kw]
            if ni < 3:
                tv = t2_ref[192 * d:192 * d + kw, :]
            else:
                tv = t2b_ref[128 * d:128 * d + kw, :]
            dd = jnp.dot(av, tv, preferred_element_type=f32)
            acc = dd if acc is None else acc + dd
        u = acc.reshape(7, 2, BB, nw)
        u = jnp.maximum(u[:, 0], u[:, 1])
        h = nw // 2
        u = jnp.maximum(u[:, :, 0:h], u[:, :, h:nw])
        outs.append(u)
    p2 = jnp.concatenate(outs, axis=2)
    p2 = jnp.maximum(p2 + b2_ref[...].reshape(1, 1, 448), 0.0)
    p2 = p2.astype(bf16)

    hacc = None
    for r in range(7):
        dd = jnp.dot(p2[r], wf1_ref[448 * r:448 * (r + 1), :],
                     preferred_element_type=f32)
        hacc = dd if hacc is None else hacc + dd
    hv = jnp.maximum(hacc + bf1_ref[...], 0.0).astype(bf16)
    o_ref[...] = jnp.dot(hv, wf2_ref[...],
                         preferred_element_type=f32) + bf2_ref[...]


@functools.partial(jax.jit, static_argnames=())
def kernel(x, w1p, b1r, w2p, b2r, wf1p, bf1r, wf2p, bf2r):
    B = x.shape[0]
    BB = _BB
    pad_b = (-B) % BB
    Bp = B + pad_b
    xi = x.reshape(B, 28, 28)
    if pad_b:
        xi = jnp.pad(xi, ((0, pad_b), (0, 0), (0, 0)))

    xh = jnp.transpose(
        jnp.pad(xi, ((0, 0), (1, 1), (1, 3))).astype(jnp.bfloat16), (1, 0, 2))


    wi1 = np.arange(32)[:, None]
    n1 = np.arange(896)[None, :]
    ni_ = n1 // 256
    loc = n1 % 256
    half = np.where(ni_ < 3, 128, 64)
    won = 2 * (4 * ni_ + (loc % half) // 32) + loc // half
    dblocks = []
    for d in range(3):
        blk = jnp.zeros((32, 896), jnp.float32)
        for kw in range(3):
            ind = jnp.asarray((wi1 == won + kw).astype(np.float32))
            rowt = jnp.broadcast_to(w1p[3 * d + kw].astype(jnp.float32),
                                    (28, 32)).reshape(896)
            blk = blk + ind * rowt[None, :]
        dblocks.append(blk)
    t1 = jnp.concatenate(dblocks, axis=0).astype(jnp.bfloat16)

    q2 = np.arange(192)[:, None]
    w2l2 = q2 // 32
    m2 = np.arange(256)[None, :]
    wol2 = 2 * ((m2 % 128) // 64) + m2 // 128
    qb = np.arange(128)[:, None]
    w2lb = qb // 32
    mb = np.arange(128)[None, :]
    wolb = mb // 64
    d2blocks = []
    dbblocks = []
    for d in range(3):
        blk = jnp.zeros((192, 256), jnp.float32)
        blkb = jnp.zeros((128, 128), jnp.float32)
        for kw in range(3):
            wsub = w2p[(3 * d + kw) * 32:(3 * d + kw) * 32 + 32].astype(
                jnp.float32)
            ind = jnp.asarray((w2l2 == wol2 + kw).astype(np.float32))
            blk = blk + ind * jnp.tile(wsub, (6, 4))
            indb = jnp.asarray((w2lb == wolb + kw).astype(np.float32))
            blkb = blkb + indb * jnp.tile(wsub, (4, 2))
        d2blocks.append(blk)
        dbblocks.append(blkb)
    t2 = jnp.concatenate(d2blocks, axis=0).astype(jnp.bfloat16)
    t2b = jnp.concatenate(dbblocks, axis=0).astype(jnp.bfloat16)

    b1t = jnp.tile(b1r, (1, 14))
    b2t = jnp.tile(b2r, (1, 7))

    nb = Bp // BB
    out = pl.pallas_call(
        _fused_body,
        out_shape=jax.ShapeDtypeStruct((Bp, 128), jnp.float32),
        grid=(nb,),
        in_specs=[
            pl.BlockSpec((30, BB, 32), lambda i: (0, i, 0)),
            pl.BlockSpec((96, 896), lambda i: (0, 0)),
            pl.BlockSpec((576, 256), lambda i: (0, 0)),
            pl.BlockSpec((384, 128), lambda i: (0, 0)),
            pl.BlockSpec((3136, 128), lambda i: (0, 0)),
            pl.BlockSpec((128, 128), lambda i: (0, 0)),
            pl.BlockSpec((1, 448), lambda i: (0, 0)),
            pl.BlockSpec((1, 448), lambda i: (0, 0)),
            pl.BlockSpec((1, 128), lambda i: (0, 0)),
            pl.BlockSpec((1, 128), lambda i: (0, 0)),
        ],
        out_specs=pl.BlockSpec((BB, 128), lambda i: (i, 0)),
        compiler_params=pltpu.CompilerParams(
            dimension_semantics=("parallel",),
            vmem_limit_bytes=_VMEM_LIMIT),
    )(xh, t1, t2, t2b, wf1p, wf2p, b1t, b2t, bf1r, bf2r)
    return out[:B, :10]


# speedup vs baseline: 1.0772x; 1.0772x over previous
import functools

import numpy as np

import jax
import jax.numpy as jnp
from jax.experimental import pallas as pl
from jax.experimental.pallas import tpu as pltpu

_BB = 128
_VMEM_LIMIT = 100 * 1024 * 1024


def _fused_body(xh_ref, t1_ref, t2_ref, t2b_ref, wf1_ref, wf2_ref,
                b1_ref, b2_ref, bf1_ref, bf2_ref, o_ref):
    BB = xh_ref.shape[1]
    M1 = 28 * BB
    M2 = 14 * BB
    f32 = jnp.float32
    bf16 = jnp.bfloat16

    a1 = jnp.concatenate(
        [xh_ref[d:d + 28].reshape(M1, 32) for d in range(3)], axis=1)
    p1b = []
    for ci in range(4):
        nw = 256 if ci < 3 else 128
        acc = jnp.dot(a1, t1_ref[:, 256 * ci:256 * ci + nw],
                      preferred_element_type=f32)
        u = acc.reshape(14, 2, BB, nw)
        u = jnp.maximum(u[:, 0], u[:, 1])
        h = nw // 2
        u = jnp.maximum(u[:, :, 0:h], u[:, :, h:nw])
        p1b.append(u)
    v = jnp.concatenate(p1b, axis=2)
    v = jnp.maximum(v + b1_ref[...].reshape(1, 1, 448), 0.0)
    p1 = v.astype(bf16)

    zc = jnp.zeros((14, BB, 32), bf16)
    zr = jnp.zeros((1, BB, 512), bf16)
    x2 = jnp.concatenate([zc, p1, zc], axis=2)
    x2 = jnp.concatenate([zr, x2, zr], axis=0)

    outs = []
    for ni in range(4):
        kw = 192 if ni < 3 else 128
        nw = 256 if ni < 3 else 128
        acc = None
        for d in range(3):
            av = x2[d:d + 14].reshape(M2, 512)[:, 128 * ni:128 * ni + kw]
            if ni < 3:
                tv = t2_ref[192 * d:192 * d + kw, :]
            else:
                tv = t2b_ref[128 * d:128 * d + kw, :]
            dd = jnp.dot(av, tv, preferred_element_type=f32)
            acc = dd if acc is None else acc + dd
        u = acc.reshape(7, 2, BB, nw)
        u = jnp.maximum(u[:, 0], u[:, 1])
        h = nw // 2
        u = jnp.maximum(u[:, :, 0:h], u[:, :, h:nw])
        outs.append(u)
    p2 = jnp.concatenate(outs, axis=2)
    p2 = jnp.maximum(p2 + b2_ref[...].reshape(1, 1, 448), 0.0)
    p2 = p2.astype(bf16)

    hacc = None
    for r in range(7):
        dd = jnp.dot(p2[r], wf1_ref[448 * r:448 * (r + 1), :],
                     preferred_element_type=f32)
        hacc = dd if hacc is None else hacc + dd
    hv = jnp.maximum(hacc + bf1_ref[...], 0.0).astype(bf16)
    o_ref[...] = jnp.dot(hv, wf2_ref[...],
                         preferred_element_type=f32) + bf2_ref[...]


@functools.partial(jax.jit, static_argnames=())
def kernel(x, w1p, b1r, w2p, b2r, wf1p, bf1r, wf2p, bf2r):
    B = x.shape[0]
    BB = _BB
    pad_b = (-B) % BB
    Bp = B + pad_b
    xi = x.reshape(B, 28, 28)
    if pad_b:
        xi = jnp.pad(xi, ((0, pad_b), (0, 0), (0, 0)))

    xh = jnp.transpose(
        jnp.pad(xi, ((0, 0), (1, 1), (1, 3))).astype(jnp.bfloat16), (1, 0, 2))


    wi1 = np.arange(32)[:, None]
    n1 = np.arange(896)[None, :]
    ni_ = n1 // 256
    loc = n1 % 256
    half = np.where(ni_ < 3, 128, 64)
    won = 2 * (4 * ni_ + (loc % half) // 32) + loc // half
    dblocks = []
    for d in range(3):
        blk = jnp.zeros((32, 896), jnp.float32)
        for kw in range(3):
            ind = jnp.asarray((wi1 == won + kw).astype(np.float32))
            rowt = jnp.broadcast_to(w1p[3 * d + kw].astype(jnp.float32),
                                    (28, 32)).reshape(896)
            blk = blk + ind * rowt[None, :]
        dblocks.append(blk)
    t1 = jnp.concatenate(dblocks, axis=0).astype(jnp.bfloat16)

    q2 = np.arange(192)[:, None]
    w2l2 = q2 // 32
    m2 = np.arange(256)[None, :]
    wol2 = 2 * ((m2 % 128) // 64) + m2 // 128
    qb = np.arange(128)[:, None]
    w2lb = qb // 32
    mb = np.arange(128)[None, :]
    wolb = mb // 64
    d2blocks = []
    dbblocks = []
    for d in range(3):
        blk = jnp.zeros((192, 256), jnp.float32)
        blkb = jnp.zeros((128, 128), jnp.float32)
        for kw in range(3):
            wsub = w2p[(3 * d + kw) * 32:(3 * d + kw) * 32 + 32].astype(
                jnp.float32)
            ind = jnp.asarray((w2l2 == wol2 + kw).astype(np.float32))
            blk = blk + ind * jnp.tile(wsub, (6, 4))
            indb = jnp.asarray((w2lb == wolb + kw).astype(np.float32))
            blkb = blkb + indb * jnp.tile(wsub, (4, 2))
        d2blocks.append(blk)
        dbblocks.append(blkb)
    t2 = jnp.concatenate(d2blocks, axis=0).astype(jnp.bfloat16)
    t2b = jnp.concatenate(dbblocks, axis=0).astype(jnp.bfloat16)

    b1t = jnp.tile(b1r, (1, 14))
    b2t = jnp.tile(b2r, (1, 7))

    nb = Bp // BB
    out = pl.pallas_call(
        _fused_body,
        out_shape=jax.ShapeDtypeStruct((Bp, 128), jnp.float32),
        grid=(nb,),
        in_specs=[
            pl.BlockSpec((30, BB, 32), lambda i: (0, i, 0)),
            pl.BlockSpec((96, 896), lambda i: (0, 0)),
            pl.BlockSpec((576, 256), lambda i: (0, 0)),
            pl.BlockSpec((384, 128), lambda i: (0, 0)),
            pl.BlockSpec((3136, 128), lambda i: (0, 0)),
            pl.BlockSpec((128, 128), lambda i: (0, 0)),
            pl.BlockSpec((1, 448), lambda i: (0, 0)),
            pl.BlockSpec((1, 448), lambda i: (0, 0)),
            pl.BlockSpec((1, 128), lambda i: (0, 0)),
            pl.BlockSpec((1, 128), lambda i: (0, 0)),
        ],
        out_specs=pl.BlockSpec((BB, 128), lambda i: (i, 0)),
        compiler_params=pltpu.CompilerParams(
            dimension_semantics=("parallel",),
            vmem_limit_bytes=_VMEM_LIMIT),
    )(xh, t1, t2, t2b, wf1p, wf2p, b1t, b2t, bf1r, bf2r)
    return out[:B, :10]


# speedup vs baseline: 1.1061x; 1.0269x over previous
import functools

import numpy as np

import jax
import jax.numpy as jnp
from jax.experimental import pallas as pl
from jax.experimental.pallas import tpu as pltpu

_BB = 256
_VMEM_LIMIT = 100 * 1024 * 1024


def _fused_body(xh_ref, t1_ref, t2_ref, t2b_ref, wf1_ref, wf2_ref,
                b1_ref, b2_ref, bf1_ref, bf2_ref, o_ref):
    BB = xh_ref.shape[1]
    M1 = 28 * BB
    M2 = 14 * BB
    f32 = jnp.float32
    bf16 = jnp.bfloat16

    a1 = jnp.concatenate(
        [xh_ref[d:d + 28].reshape(M1, 32) for d in range(3)], axis=1)
    p1b = []
    for ci in range(4):
        nw = 256 if ci < 3 else 128
        acc = jnp.dot(a1, t1_ref[:, 256 * ci:256 * ci + nw],
                      preferred_element_type=f32)
        u = acc.reshape(14, 2, BB, nw)
        u = jnp.maximum(u[:, 0], u[:, 1])
        h = nw // 2
        u = jnp.maximum(u[:, :, 0:h], u[:, :, h:nw])
        p1b.append(u)
    v = jnp.concatenate(p1b, axis=2)
    v = jnp.maximum(v + b1_ref[...].reshape(1, 1, 448), 0.0)
    p1 = v.astype(bf16)

    zc = jnp.zeros((14, BB, 32), bf16)
    zr = jnp.zeros((1, BB, 512), bf16)
    x2 = jnp.concatenate([zc, p1, zc], axis=2)
    x2 = jnp.concatenate([zr, x2, zr], axis=0)

    outs = []
    for ni in range(4):
        kw = 192 if ni < 3 else 128
        nw = 256 if ni < 3 else 128
        acc = None
        for d in range(3):
            av = x2[d:d + 14].reshape(M2, 512)[:, 128 * ni:128 * ni + kw]
            if ni < 3:
                tv = t2_ref[192 * d:192 * d + kw, :]
            else:
                tv = t2b_ref[128 * d:128 * d + kw, :]
            dd = jnp.dot(av, tv, preferred_element_type=f32)
            acc = dd if acc is None else acc + dd
        u = acc.reshape(7, 2, BB, nw)
        u = jnp.maximum(u[:, 0], u[:, 1])
        h = nw // 2
        u = jnp.maximum(u[:, :, 0:h], u[:, :, h:nw])
        outs.append(u)
    p2 = jnp.concatenate(outs, axis=2)
    p2 = jnp.maximum(p2 + b2_ref[...].reshape(1, 1, 448), 0.0)
    p2 = p2.astype(bf16)

    hacc = None
    for r in range(7):
        dd = jnp.dot(p2[r], wf1_ref[448 * r:448 * (r + 1), :],
                     preferred_element_type=f32)
        hacc = dd if hacc is None else hacc + dd
    hv = jnp.maximum(hacc + bf1_ref[...], 0.0).astype(bf16)
    o_ref[...] = jnp.dot(hv, wf2_ref[...],
                         preferred_element_type=f32) + bf2_ref[...]


@functools.partial(jax.jit, static_argnames=())
def kernel(x, w1p, b1r, w2p, b2r, wf1p, bf1r, wf2p, bf2r):
    B = x.shape[0]
    BB = _BB
    pad_b = (-B) % BB
    Bp = B + pad_b
    xi = x.reshape(B, 28, 28)
    if pad_b:
        xi = jnp.pad(xi, ((0, pad_b), (0, 0), (0, 0)))

    xh = jnp.transpose(
        jnp.pad(xi, ((0, 0), (1, 1), (1, 3))).astype(jnp.bfloat16), (1, 0, 2))


    wi1 = np.arange(32)[:, None]
    n1 = np.arange(896)[None, :]
    ni_ = n1 // 256
    loc = n1 % 256
    half = np.where(ni_ < 3, 128, 64)
    won = 2 * (4 * ni_ + (loc % half) // 32) + loc // half
    dblocks = []
    for d in range(3):
        blk = jnp.zeros((32, 896), jnp.float32)
        for kw in range(3):
            ind = jnp.asarray((wi1 == won + kw).astype(np.float32))
            rowt = jnp.broadcast_to(w1p[3 * d + kw].astype(jnp.float32),
                                    (28, 32)).reshape(896)
            blk = blk + ind * rowt[None, :]
        dblocks.append(blk)
    t1 = jnp.concatenate(dblocks, axis=0).astype(jnp.bfloat16)

    q2 = np.arange(192)[:, None]
    w2l2 = q2 // 32
    m2 = np.arange(256)[None, :]
    wol2 = 2 * ((m2 % 128) // 64) + m2 // 128
    qb = np.arange(128)[:, None]
    w2lb = qb // 32
    mb = np.arange(128)[None, :]
    wolb = mb // 64
    d2blocks = []
    dbblocks = []
    for d in range(3):
        blk = jnp.zeros((192, 256), jnp.float32)
        blkb = jnp.zeros((128, 128), jnp.float32)
        for kw in range(3):
            wsub = w2p[(3 * d + kw) * 32:(3 * d + kw) * 32 + 32].astype(
                jnp.float32)
            ind = jnp.asarray((w2l2 == wol2 + kw).astype(np.float32))
            blk = blk + ind * jnp.tile(wsub, (6, 4))
            indb = jnp.asarray((w2lb == wolb + kw).astype(np.float32))
            blkb = blkb + indb * jnp.tile(wsub, (4, 2))
        d2blocks.append(blk)
        dbblocks.append(blkb)
    t2 = jnp.concatenate(d2blocks, axis=0).astype(jnp.bfloat16)
    t2b = jnp.concatenate(dbblocks, axis=0).astype(jnp.bfloat16)

    b1t = jnp.tile(b1r, (1, 14))
    b2t = jnp.tile(b2r, (1, 7))

    nb = Bp // BB
    out = pl.pallas_call(
        _fused_body,
        out_shape=jax.ShapeDtypeStruct((Bp, 128), jnp.float32),
        grid=(nb,),
        in_specs=[
            pl.BlockSpec((30, BB, 32), lambda i: (0, i, 0)),
            pl.BlockSpec((96, 896), lambda i: (0, 0)),
            pl.BlockSpec((576, 256), lambda i: (0, 0)),
            pl.BlockSpec((384, 128), lambda i: (0, 0)),
            pl.BlockSpec((3136, 128), lambda i: (0, 0)),
            pl.BlockSpec((128, 128), lambda i: (0, 0)),
            pl.BlockSpec((1, 448), lambda i: (0, 0)),
            pl.BlockSpec((1, 448), lambda i: (0, 0)),
            pl.BlockSpec((1, 128), lambda i: (0, 0)),
            pl.BlockSpec((1, 128), lambda i: (0, 0)),
        ],
        out_specs=pl.BlockSpec((BB, 128), lambda i: (i, 0)),
        compiler_params=pltpu.CompilerParams(
            dimension_semantics=("parallel",),
            vmem_limit_bytes=_VMEM_LIMIT),
    )(xh, t1, t2, t2b, wf1p, wf2p, b1t, b2t, bf1r, bf2r)
    return out[:B, :10]


# speedup vs baseline: 1.1223x; 1.0147x over previous
import functools

import numpy as np

import jax
import jax.numpy as jnp
from jax.experimental import pallas as pl
from jax.experimental.pallas import tpu as pltpu

_BB = 512
_VMEM_LIMIT = 100 * 1024 * 1024


def _fused_body(xh_ref, t1_ref, t2_ref, t2b_ref, wf1_ref, wf2_ref,
                b1_ref, b2_ref, bf1_ref, bf2_ref, o_ref):
    BB = xh_ref.shape[1]
    M1 = 28 * BB
    M2 = 14 * BB
    f32 = jnp.float32
    bf16 = jnp.bfloat16

    a1 = jnp.concatenate(
        [xh_ref[d:d + 28].reshape(M1, 32) for d in range(3)], axis=1)
    p1b = []
    for ci in range(4):
        nw = 256 if ci < 3 else 128
        acc = jnp.dot(a1, t1_ref[:, 256 * ci:256 * ci + nw],
                      preferred_element_type=f32)
        u = acc.reshape(14, 2, BB, nw)
        u = jnp.maximum(u[:, 0], u[:, 1])
        h = nw // 2
        u = jnp.maximum(u[:, :, 0:h], u[:, :, h:nw])
        p1b.append(u)
    v = jnp.concatenate(p1b, axis=2)
    v = jnp.maximum(v + b1_ref[...].reshape(1, 1, 448), 0.0)
    p1 = v.astype(bf16)

    zc = jnp.zeros((14, BB, 32), bf16)
    zr = jnp.zeros((1, BB, 512), bf16)
    x2 = jnp.concatenate([zc, p1, zc], axis=2)
    x2 = jnp.concatenate([zr, x2, zr], axis=0)

    outs = []
    for ni in range(4):
        kw = 192 if ni < 3 else 128
        nw = 256 if ni < 3 else 128
        acc = None
        for d in range(3):
            av = x2[d:d + 14].reshape(M2, 512)[:, 128 * ni:128 * ni + kw]
            if ni < 3:
                tv = t2_ref[192 * d:192 * d + kw, :]
            else:
                tv = t2b_ref[128 * d:128 * d + kw, :]
            dd = jnp.dot(av, tv, preferred_element_type=f32)
            acc = dd if acc is None else acc + dd
        u = acc.reshape(7, 2, BB, nw)
        u = jnp.maximum(u[:, 0], u[:, 1])
        h = nw // 2
        u = jnp.maximum(u[:, :, 0:h], u[:, :, h:nw])
        outs.append(u)
    p2 = jnp.concatenate(outs, axis=2)
    p2 = jnp.maximum(p2 + b2_ref[...].reshape(1, 1, 448), 0.0)
    p2 = p2.astype(bf16)

    hacc = None
    for r in range(7):
        dd = jnp.dot(p2[r], wf1_ref[448 * r:448 * (r + 1), :],
                     preferred_element_type=f32)
        hacc = dd if hacc is None else hacc + dd
    hv = jnp.maximum(hacc + bf1_ref[...], 0.0).astype(bf16)
    o_ref[...] = jnp.dot(hv, wf2_ref[...],
                         preferred_element_type=f32) + bf2_ref[...]


@functools.partial(jax.jit, static_argnames=())
def kernel(x, w1p, b1r, w2p, b2r, wf1p, bf1r, wf2p, bf2r):
    B = x.shape[0]
    BB = _BB
    pad_b = (-B) % BB
    Bp = B + pad_b
    xi = x.reshape(B, 28, 28)
    if pad_b:
        xi = jnp.pad(xi, ((0, pad_b), (0, 0), (0, 0)))

    xh = jnp.transpose(
        jnp.pad(xi, ((0, 0), (1, 1), (1, 3))).astype(jnp.bfloat16), (1, 0, 2))


    wi1 = np.arange(32)[:, None]
    n1 = np.arange(896)[None, :]
    ni_ = n1 // 256
    loc = n1 % 256
    half = np.where(ni_ < 3, 128, 64)
    won = 2 * (4 * ni_ + (loc % half) // 32) + loc // half
    dblocks = []
    for d in range(3):
        blk = jnp.zeros((32, 896), jnp.float32)
        for kw in range(3):
            ind = jnp.asarray((wi1 == won + kw).astype(np.float32))
            rowt = jnp.broadcast_to(w1p[3 * d + kw].astype(jnp.float32),
                                    (28, 32)).reshape(896)
            blk = blk + ind * rowt[None, :]
        dblocks.append(blk)
    t1 = jnp.concatenate(dblocks, axis=0).astype(jnp.bfloat16)

    q2 = np.arange(192)[:, None]
    w2l2 = q2 // 32
    m2 = np.arange(256)[None, :]
    wol2 = 2 * ((m2 % 128) // 64) + m2 // 128
    qb = np.arange(128)[:, None]
    w2lb = qb // 32
    mb = np.arange(128)[None, :]
    wolb = mb // 64
    d2blocks = []
    dbblocks = []
    for d in range(3):
        blk = jnp.zeros((192, 256), jnp.float32)
        blkb = jnp.zeros((128, 128), jnp.float32)
        for kw in range(3):
            wsub = w2p[(3 * d + kw) * 32:(3 * d + kw) * 32 + 32].astype(
                jnp.float32)
            ind = jnp.asarray((w2l2 == wol2 + kw).astype(np.float32))
            blk = blk + ind * jnp.tile(wsub, (6, 4))
            indb = jnp.asarray((w2lb == wolb + kw).astype(np.float32))
            blkb = blkb + indb * jnp.tile(wsub, (4, 2))
        d2blocks.append(blk)
        dbblocks.append(blkb)
    t2 = jnp.concatenate(d2blocks, axis=0).astype(jnp.bfloat16)
    t2b = jnp.concatenate(dbblocks, axis=0).astype(jnp.bfloat16)

    b1t = jnp.tile(b1r, (1, 14))
    b2t = jnp.tile(b2r, (1, 7))

    nb = Bp // BB
    out = pl.pallas_call(
        _fused_body,
        out_shape=jax.ShapeDtypeStruct((Bp, 128), jnp.float32),
        grid=(nb,),
        in_specs=[
            pl.BlockSpec((30, BB, 32), lambda i: (0, i, 0)),
            pl.BlockSpec((96, 896), lambda i: (0, 0)),
            pl.BlockSpec((576, 256), lambda i: (0, 0)),
            pl.BlockSpec((384, 128), lambda i: (0, 0)),
            pl.BlockSpec((3136, 128), lambda i: (0, 0)),
            pl.BlockSpec((128, 128), lambda i: (0, 0)),
            pl.BlockSpec((1, 448), lambda i: (0, 0)),
            pl.BlockSpec((1, 448), lambda i: (0, 0)),
            pl.BlockSpec((1, 128), lambda i: (0, 0)),
            pl.BlockSpec((1, 128), lambda i: (0, 0)),
        ],
        out_specs=pl.BlockSpec((BB, 128), lambda i: (i, 0)),
        compiler_params=pltpu.CompilerParams(
            dimension_semantics=("parallel",),
            vmem_limit_bytes=_VMEM_LIMIT),
    )(xh, t1, t2, t2b, wf1p, wf2p, b1t, b2t, bf1r, bf2r)
    return out[:B, :10]
